# baseline (speedup 1.0000x reference)
import jax
import jax.numpy as jnp
from jax.experimental import pallas as pl
from jax.experimental.pallas import tpu as pltpu

_NUM_NODES = 64
_D_FEAT = 256
_D_HID = 1024
_D_OUT = 1024
_BATCH = 4096
_BLOCK = 2048


def _mlp_bcast_kernel(emb_ref, w1_ref, b1_ref, w2_ref, b2_ref, out_ref, g_ref):
    @pl.when(pl.program_id(0) == 0)
    def _compute_g():
        h = jnp.dot(emb_ref[:], w1_ref[:], preferred_element_type=jnp.float32)
        h = jnp.maximum(h + b1_ref[:], 0.0)
        hbar = jnp.mean(h, axis=0, keepdims=True)
        g = jnp.dot(hbar, w2_ref[:], preferred_element_type=jnp.float32)
        g_ref[:] = g + b2_ref[:]
    out_ref[:] = jnp.broadcast_to(g_ref[:], (_BLOCK, _D_OUT))


def kernel(emb, W1, b1, W2, b2, batch_size):
    del batch_size
    b1r = b1.reshape(1, _D_HID)
    b2r = b2.reshape(1, _D_OUT)
    grid = (_BATCH // _BLOCK,)
    return pl.pallas_call(
        _mlp_bcast_kernel,
        grid=grid,
        in_specs=[
            pl.BlockSpec((_NUM_NODES, _D_FEAT), lambda i: (0, 0)),
            pl.BlockSpec((_D_FEAT, _D_HID), lambda i: (0, 0)),
            pl.BlockSpec((1, _D_HID), lambda i: (0, 0)),
            pl.BlockSpec((_D_HID, _D_OUT), lambda i: (0, 0)),
            pl.BlockSpec((1, _D_OUT), lambda i: (0, 0)),
        ],
        out_specs=pl.BlockSpec((_BLOCK, _D_OUT), lambda i: (i, 0)),
        out_shape=jax.ShapeDtypeStruct((_BATCH, _D_OUT), jnp.float32),
        scratch_shapes=[pltpu.VMEM((1, _D_OUT), jnp.float32)],
    )(emb, W1, b1r, W2, b2r)


# speedup vs baseline: 1.1326x; 1.1326x over previous
import jax
import jax.numpy as jnp
from jax.experimental import pallas as pl
from jax.experimental.pallas import tpu as pltpu

_NUM_NODES = 64
_D_FEAT = 256
_D_HID = 1024
_D_OUT = 1024
_BATCH = 4096
_BLOCK = 1024


def _mlp_bcast_kernel(emb_ref, w1_ref, b1_ref, w2_ref, b2_ref, out_ref, g_ref):
    @pl.when(pl.program_id(0) == 0)
    def _compute_g():
        h = jnp.dot(emb_ref[:], w1_ref[:], preferred_element_type=jnp.float32)
        h = jnp.maximum(h + b1_ref[:], 0.0)
        hbar = jnp.mean(h, axis=0, keepdims=True)
        g = jnp.dot(hbar, w2_ref[:], preferred_element_type=jnp.float32)
        g_ref[:] = g + b2_ref[:]
    out_ref[:] = jnp.broadcast_to(g_ref[:], (_BLOCK, _D_OUT))


def kernel(emb, W1, b1, W2, b2, batch_size):
    del batch_size
    b1r = b1.reshape(1, _D_HID)
    b2r = b2.reshape(1, _D_OUT)
    grid = (_BATCH // _BLOCK,)
    return pl.pallas_call(
        _mlp_bcast_kernel,
        grid=grid,
        in_specs=[
            pl.BlockSpec((_NUM_NODES, _D_FEAT), lambda i: (0, 0)),
            pl.BlockSpec((_D_FEAT, _D_HID), lambda i: (0, 0)),
            pl.BlockSpec((1, _D_HID), lambda i: (0, 0)),
            pl.BlockSpec((_D_HID, _D_OUT), lambda i: (0, 0)),
            pl.BlockSpec((1, _D_OUT), lambda i: (0, 0)),
        ],
        out_specs=pl.BlockSpec((_BLOCK, _D_OUT), lambda i: (i, 0)),
        out_shape=jax.ShapeDtypeStruct((_BATCH, _D_OUT), jnp.float32),
        scratch_shapes=[pltpu.VMEM((1, _D_OUT), jnp.float32)],
    )(emb, W1, b1r, W2, b2r)
